# Initial kernel scaffold; baseline (speedup 1.0000x reference)
import jax
import jax.numpy as jnp
from jax.experimental import pallas as pl


def kernel(x, edge_index, W1, b1, W2, b2, W3, b3):
    raise NotImplementedError("write your pallas kernel here")


# speedup vs baseline: 17.9957x; 17.9957x over previous
import functools

import jax
import jax.numpy as jnp
from jax import lax
from jax.experimental import pallas as pl
from jax.experimental.pallas import tpu as pltpu
from jax.experimental.pallas import tpu_sc as plsc

_N = 10000
_E = 320000
_K = 80
_NC = 2
_NS = 16
_PER_TILE = _E // (_NC * _NS)
_CH = _PER_TILE // _K
_NPAD = 10240
_ZROWS = _NPAD // _NS


def _make_sc_aggregate(D, tc_tiling=True):
    mesh = plsc.VectorSubcoreMesh(core_axis_name="c", subcore_axis_name="s")

    @functools.partial(
        pl.kernel,
        out_type=jax.ShapeDtypeStruct((_NC, _NPAD, D), jnp.float32),
        mesh=mesh,
        compiler_params=pltpu.CompilerParams(use_tc_tiling_on_sc=tc_tiling),
        scratch_types=[
            pltpu.VMEM((_CH, _K), jnp.int32),
            pltpu.VMEM((_CH, _K), jnp.int32),
            pltpu.VMEM((_K, D), jnp.float32),
            pltpu.VMEM_SHARED((_NPAD, D), jnp.float32),
        ],
    )
    def agg(p_hbm, src_hbm, dst_hbm, zeros_hbm, out_hbm, src_i, dst_i, rows, acc):
        c = lax.axis_index("c")
        s = lax.axis_index("s")

        pltpu.sync_copy(zeros_hbm, acc.at[pl.ds(s * _ZROWS, _ZROWS)])

        t = c * _NS + s
        pltpu.sync_copy(src_hbm.at[t], src_i)
        pltpu.sync_copy(dst_hbm.at[t], dst_i)

        plsc.subcore_barrier()

        def body(j, carry):
            pltpu.sync_copy(p_hbm.at[src_i.at[j]], rows)
            pltpu.sync_copy(rows, acc.at[dst_i.at[j]], add=True)
            return carry

        lax.fori_loop(0, _CH, body, 0)

        plsc.subcore_barrier()

        r0 = s * _ZROWS
        pltpu.sync_copy(acc.at[pl.ds(r0, _ZROWS)],
                        out_hbm.at[c, pl.ds(r0, _ZROWS)])

    return agg


def _make_sc_degree():
    mesh = plsc.VectorSubcoreMesh(core_axis_name="c", subcore_axis_name="s")

    @functools.partial(
        pl.kernel,
        out_type=jax.ShapeDtypeStruct((_NC, _NPAD), jnp.float32),
        mesh=mesh,
        scratch_types=[
            pltpu.VMEM((_CH, _K), jnp.int32),
            pltpu.VMEM((_K,), jnp.float32),
            pltpu.VMEM((_ZROWS,), jnp.float32),
            pltpu.VMEM_SHARED((_NPAD,), jnp.float32),
        ],
    )
    def deg(dst_hbm, zeros_hbm, out_hbm, dst_i, ones_v, zv, acc):
        c = lax.axis_index("c")
        s = lax.axis_index("s")
        pltpu.sync_copy(zeros_hbm, zv)
        pltpu.sync_copy(zv, acc.at[pl.ds(s * _ZROWS, _ZROWS)])
        for i in range(_K // 16):
            ones_v[pl.ds(i * 16, 16)] = jnp.ones((16,), jnp.float32)
        t = c * _NS + s
        pltpu.sync_copy(dst_hbm.at[t], dst_i)
        plsc.subcore_barrier()

        def body(j, carry):
            pltpu.sync_copy(ones_v, acc.at[dst_i.at[j]], add=True)
            return carry

        lax.fori_loop(0, _CH, body, 0)
        plsc.subcore_barrier()
        r0 = s * _ZROWS
        pltpu.sync_copy(acc.at[pl.ds(r0, _ZROWS)], out_hbm.at[c, pl.ds(r0, _ZROWS)])

    return deg


_sc_agg_128 = _make_sc_aggregate(128)
_sc_agg_64 = _make_sc_aggregate(64, tc_tiling=False)
_sc_deg = _make_sc_degree()

_BLK = 2000


def _tc_first(degT, x, W1):
    def body(deg_ref, x_ref, w_ref, dis_ref, p_ref):
        deg = deg_ref[:, 0:1] + deg_ref[:, 1:2] + 1.0
        d = lax.rsqrt(deg)
        dis_ref[...] = d
        p_ref[...] = d * jnp.dot(x_ref[...], w_ref[...],
                                 preferred_element_type=jnp.float32)

    return pl.pallas_call(
        body,
        grid=(_N // _BLK,),
        in_specs=[
            pl.BlockSpec((_BLK, 2), lambda i: (i, 0)),
            pl.BlockSpec((_BLK, 128), lambda i: (i, 0)),
            pl.BlockSpec((128, 128), lambda i: (0, 0)),
        ],
        out_specs=[
            pl.BlockSpec((_BLK, 1), lambda i: (i, 0)),
            pl.BlockSpec((_BLK, 128), lambda i: (i, 0)),
        ],
        out_shape=[
            jax.ShapeDtypeStruct((_N, 1), jnp.float32),
            jax.ShapeDtypeStruct((_N, 128), jnp.float32),
        ],
    )(degT, x, W1)


def _tc_mid(S, P, dis, b, W, D, Dn):
    def body(s_ref, p_ref, dis_ref, b_ref, w_ref, out_ref):
        d = dis_ref[...]
        h = d * (s_ref[0] + s_ref[1] + p_ref[...]) + b_ref[...]
        h = jnp.maximum(h, 0.0)
        out_ref[...] = d * jnp.dot(h, w_ref[...],
                                   preferred_element_type=jnp.float32)

    return pl.pallas_call(
        body,
        grid=(_N // _BLK,),
        in_specs=[
            pl.BlockSpec((2, _BLK, D), lambda i: (0, i, 0)),
            pl.BlockSpec((_BLK, D), lambda i: (i, 0)),
            pl.BlockSpec((_BLK, 1), lambda i: (i, 0)),
            pl.BlockSpec((1, D), lambda i: (0, 0)),
            pl.BlockSpec((D, Dn), lambda i: (0, 0)),
        ],
        out_specs=pl.BlockSpec((_BLK, Dn), lambda i: (i, 0)),
        out_shape=jax.ShapeDtypeStruct((_N, Dn), jnp.float32),
    )(S, P, dis, b, W)


def _tc_final(S, P, dis, b, D):
    def body(s_ref, p_ref, dis_ref, b_ref, out_ref):
        d = dis_ref[...]
        h = d * (s_ref[0] + s_ref[1] + p_ref[...]) + b_ref[...]
        m = jnp.max(h, axis=1, keepdims=True)
        sh = h - m
        out_ref[...] = sh - jnp.log(jnp.sum(jnp.exp(sh), axis=1, keepdims=True))

    return pl.pallas_call(
        body,
        grid=(_N // _BLK,),
        in_specs=[
            pl.BlockSpec((2, _BLK, D), lambda i: (0, i, 0)),
            pl.BlockSpec((_BLK, D), lambda i: (i, 0)),
            pl.BlockSpec((_BLK, 1), lambda i: (i, 0)),
            pl.BlockSpec((1, D), lambda i: (0, 0)),
        ],
        out_specs=pl.BlockSpec((_BLK, D), lambda i: (i, 0)),
        out_shape=jax.ShapeDtypeStruct((_N, D), jnp.float32),
    )(S, P, dis, b)


def kernel(x, edge_index, W1, b1, W2, b2, W3, b3):
    src = edge_index[0].reshape(_NC * _NS, _CH, _K)
    dst = edge_index[1].reshape(_NC * _NS, _CH, _K)
    zeros128 = jnp.zeros((_ZROWS, 128), jnp.float32)
    zeros64 = jnp.zeros((_ZROWS, 64), jnp.float32)
    zeros1d = jnp.zeros((_ZROWS,), jnp.float32)

    degS = _sc_deg(dst, zeros1d)
    dis, P1 = _tc_first(degS.T, x, W1)
    S1 = _sc_agg_128(P1, src, dst, zeros128)
    P2 = _tc_mid(S1, P1, dis, b1.reshape(1, 128), W2, 128, 128)
    S2 = _sc_agg_128(P2, src, dst, zeros128)
    P3 = _tc_mid(S2, P2, dis, b2.reshape(1, 128), W3, 128, 64)
    S3 = _sc_agg_64(P3, src, dst, zeros64)
    return _tc_final(S3, P3, dis, b3.reshape(1, 64), 64)


# speedup vs baseline: 27.2799x; 1.5159x over previous
import functools

import jax
import jax.numpy as jnp
from jax import lax
from jax.experimental import pallas as pl
from jax.experimental.pallas import tpu as pltpu
from jax.experimental.pallas import tpu_sc as plsc

_N = 10000
_E = 320000
_K = 80
_NC = 2
_NS = 16
_NPAD = 10240
_ZROWS = _NPAD // _NS


def _make_sc_agg_colsplit(NBUF=4):
    mesh = plsc.VectorSubcoreMesh(core_axis_name="c", subcore_axis_name="s")
    pt = _E // _NS
    ch = pt // _K
    ng = ch // NBUF
    tail = ch - ng * NBUF

    @functools.partial(
        pl.kernel,
        out_type=jax.ShapeDtypeStruct((_NC, _NPAD, 64), jnp.float32),
        mesh=mesh,
        compiler_params=pltpu.CompilerParams(use_tc_tiling_on_sc=False),
        scratch_types=[
            pltpu.VMEM((ch, _K), jnp.int32),
            pltpu.VMEM((ch, _K), jnp.int32),
            pltpu.VMEM((NBUF, _K, 64), jnp.float32),
            pltpu.VMEM_SHARED((_NPAD, 64), jnp.float32),
            pltpu.SemaphoreType.DMA((NBUF,)),
            pltpu.SemaphoreType.DMA((NBUF,)),
        ],
    )
    def agg(p_hbm, src_hbm, dst_hbm, zeros_hbm, out_hbm, src_i, dst_i, rows, acc, gsem, ssem):
        c = lax.axis_index("c")
        s = lax.axis_index("s")
        pltpu.sync_copy(zeros_hbm, acc.at[pl.ds(s * _ZROWS, _ZROWS)])
        pltpu.sync_copy(src_hbm.at[s], src_i)
        pltpu.sync_copy(dst_hbm.at[s], dst_i)
        plsc.subcore_barrier()
        pc = p_hbm.at[c]

        def gather(j, b):
            pltpu.async_copy(pc.at[src_i.at[j]], rows.at[b], gsem.at[b])

        def wait_gather(j, b):
            pltpu.make_async_copy(pc.at[src_i.at[j]], rows.at[b], gsem.at[b]).wait()

        def scatter(j, b):
            pltpu.async_copy(rows.at[b], acc.at[dst_i.at[j]], ssem.at[b], add=True)

        def wait_scatter(j, b):
            pltpu.make_async_copy(rows.at[b], acc.at[dst_i.at[j]], ssem.at[b]).wait()

        for b in range(NBUF):
            gather(b, b)

        def group(g, carry):
            for b in range(NBUF):
                j = g * NBUF + b
                wait_gather(j, b)
                scatter(j, b)
            for b in range(NBUF):
                j = g * NBUF + b
                wait_scatter(j, b)
                jn = (g + 1) * NBUF + b

                @pl.when(jn < ch)
                def _():
                    gather(jn, b)
            return carry

        lax.fori_loop(0, ng, group, 0)
        for b in range(tail):
            j = ng * NBUF + b
            wait_gather(j, b)
            scatter(j, b)
            wait_scatter(j, b)

        plsc.subcore_barrier()
        r0 = s * _ZROWS
        pltpu.sync_copy(acc.at[pl.ds(r0, _ZROWS)], out_hbm.at[c, pl.ds(r0, _ZROWS)])

    return agg


def _make_sc_agg_edgesplit64(NBUF=4):
    mesh = plsc.VectorSubcoreMesh(core_axis_name="c", subcore_axis_name="s")
    pt = _E // (_NC * _NS)
    ch = pt // _K
    ng = ch // NBUF
    tail = ch - ng * NBUF

    @functools.partial(
        pl.kernel,
        out_type=jax.ShapeDtypeStruct((_NC, _NPAD, 64), jnp.float32),
        mesh=mesh,
        compiler_params=pltpu.CompilerParams(use_tc_tiling_on_sc=False),
        scratch_types=[
            pltpu.VMEM((ch, _K), jnp.int32),
            pltpu.VMEM((ch, _K), jnp.int32),
            pltpu.VMEM((NBUF, _K, 64), jnp.float32),
            pltpu.VMEM_SHARED((_NPAD, 64), jnp.float32),
            pltpu.SemaphoreType.DMA((NBUF,)),
            pltpu.SemaphoreType.DMA((NBUF,)),
        ],
    )
    def agg(p_hbm, src_hbm, dst_hbm, zeros_hbm, out_hbm, src_i, dst_i, rows, acc, gsem, ssem):
        c = lax.axis_index("c")
        s = lax.axis_index("s")
        pltpu.sync_copy(zeros_hbm, acc.at[pl.ds(s * _ZROWS, _ZROWS)])
        t = c * _NS + s
        pltpu.sync_copy(src_hbm.at[t], src_i)
        pltpu.sync_copy(dst_hbm.at[t], dst_i)
        plsc.subcore_barrier()

        def gather(j, b):
            pltpu.async_copy(p_hbm.at[src_i.at[j]], rows.at[b], gsem.at[b])

        def wait_gather(j, b):
            pltpu.make_async_copy(p_hbm.at[src_i.at[j]], rows.at[b], gsem.at[b]).wait()

        def scatter(j, b):
            pltpu.async_copy(rows.at[b], acc.at[dst_i.at[j]], ssem.at[b], add=True)

        def wait_scatter(j, b):
            pltpu.make_async_copy(rows.at[b], acc.at[dst_i.at[j]], ssem.at[b]).wait()

        for b in range(NBUF):
            gather(b, b)

        def group(g, carry):
            for b in range(NBUF):
                j = g * NBUF + b
                wait_gather(j, b)
                scatter(j, b)
            for b in range(NBUF):
                j = g * NBUF + b
                wait_scatter(j, b)
                jn = (g + 1) * NBUF + b

                @pl.when(jn < ch)
                def _():
                    gather(jn, b)
            return carry

        lax.fori_loop(0, ng, group, 0)
        for b in range(tail):
            j = ng * NBUF + b
            wait_gather(j, b)
            scatter(j, b)
            wait_scatter(j, b)

        plsc.subcore_barrier()
        r0 = s * _ZROWS
        pltpu.sync_copy(acc.at[pl.ds(r0, _ZROWS)], out_hbm.at[c, pl.ds(r0, _ZROWS)])

    return agg


def _make_sc_degree():
    mesh = plsc.VectorSubcoreMesh(core_axis_name="c", subcore_axis_name="s")
    pt = _E // (_NC * _NS)
    ch = pt // _K

    @functools.partial(
        pl.kernel,
        out_type=jax.ShapeDtypeStruct((_NC, _NPAD), jnp.float32),
        mesh=mesh,
        scratch_types=[
            pltpu.VMEM((ch, _K), jnp.int32),
            pltpu.VMEM((_K,), jnp.float32),
            pltpu.VMEM((_ZROWS,), jnp.float32),
            pltpu.VMEM_SHARED((_NPAD,), jnp.float32),
        ],
    )
    def deg(dst_hbm, zeros_hbm, out_hbm, dst_i, ones_v, zv, acc):
        c = lax.axis_index("c")
        s = lax.axis_index("s")
        pltpu.sync_copy(zeros_hbm, zv)
        pltpu.sync_copy(zv, acc.at[pl.ds(s * _ZROWS, _ZROWS)])
        for i in range(_K // 16):
            ones_v[pl.ds(i * 16, 16)] = jnp.ones((16,), jnp.float32)
        t = c * _NS + s
        pltpu.sync_copy(dst_hbm.at[t], dst_i)
        plsc.subcore_barrier()

        def body(j, carry):
            pltpu.sync_copy(ones_v, acc.at[dst_i.at[j]], add=True)
            return carry

        lax.fori_loop(0, ch, body, 0)
        plsc.subcore_barrier()
        r0 = s * _ZROWS
        pltpu.sync_copy(acc.at[pl.ds(r0, _ZROWS)], out_hbm.at[c, pl.ds(r0, _ZROWS)])

    return deg


_sc_agg_cs = _make_sc_agg_colsplit()
_sc_agg_64 = _make_sc_agg_edgesplit64()
_sc_deg = _make_sc_degree()

_BLK = 2000


def _tc_first(degT, x, W1):
    def body(deg_ref, x_ref, w_ref, dis_ref, p_ref):
        deg = deg_ref[:, 0:1] + deg_ref[:, 1:2] + 1.0
        d = lax.rsqrt(deg)
        dis_ref[...] = d
        h = jnp.dot(x_ref[...], w_ref[...], preferred_element_type=jnp.float32)
        p_ref[0] = d * h[:, :64]
        p_ref[1] = d * h[:, 64:]

    return pl.pallas_call(
        body,
        grid=(_N // _BLK,),
        in_specs=[
            pl.BlockSpec((_BLK, 2), lambda i: (i, 0)),
            pl.BlockSpec((_BLK, 128), lambda i: (i, 0)),
            pl.BlockSpec((128, 128), lambda i: (0, 0)),
        ],
        out_specs=[
            pl.BlockSpec((_BLK, 1), lambda i: (i, 0)),
            pl.BlockSpec((2, _BLK, 64), lambda i: (0, i, 0)),
        ],
        out_shape=[
            jax.ShapeDtypeStruct((_N, 1), jnp.float32),
            jax.ShapeDtypeStruct((2, _N, 64), jnp.float32),
        ],
    )(degT, x, W1)


def _tc_mid(S, P, dis, b, W, Dn, split_out):
    def body(s_ref, p_ref, dis_ref, b_ref, w_ref, out_ref):
        d = dis_ref[...]
        h0 = jnp.maximum(d * (s_ref[0] + p_ref[0]) + b_ref[:, :64], 0.0)
        h1 = jnp.maximum(d * (s_ref[1] + p_ref[1]) + b_ref[:, 64:], 0.0)
        hw = (jnp.dot(h0, w_ref[:64], preferred_element_type=jnp.float32)
              + jnp.dot(h1, w_ref[64:], preferred_element_type=jnp.float32))
        if split_out:
            out_ref[0] = d * hw[:, : Dn // 2]
            out_ref[1] = d * hw[:, Dn // 2:]
        else:
            out_ref[...] = d * hw

    if split_out:
        out_spec = pl.BlockSpec((2, _BLK, Dn // 2), lambda i: (0, i, 0))
        out_shape = jax.ShapeDtypeStruct((2, _N, Dn // 2), jnp.float32)
    else:
        out_spec = pl.BlockSpec((_BLK, Dn), lambda i: (i, 0))
        out_shape = jax.ShapeDtypeStruct((_N, Dn), jnp.float32)

    return pl.pallas_call(
        body,
        grid=(_N // _BLK,),
        in_specs=[
            pl.BlockSpec((2, _BLK, 64), lambda i: (0, i, 0)),
            pl.BlockSpec((2, _BLK, 64), lambda i: (0, i, 0)),
            pl.BlockSpec((_BLK, 1), lambda i: (i, 0)),
            pl.BlockSpec((1, 128), lambda i: (0, 0)),
            pl.BlockSpec((128, Dn), lambda i: (0, 0)),
        ],
        out_specs=out_spec,
        out_shape=out_shape,
    )(S, P, dis, b, W)


def _tc_final(S, P, dis, b):
    def body(s_ref, p_ref, dis_ref, b_ref, out_ref):
        d = dis_ref[...]
        h = d * (s_ref[0] + s_ref[1] + p_ref[...]) + b_ref[...]
        m = jnp.max(h, axis=1, keepdims=True)
        sh = h - m
        out_ref[...] = sh - jnp.log(jnp.sum(jnp.exp(sh), axis=1, keepdims=True))

    return pl.pallas_call(
        body,
        grid=(_N // _BLK,),
        in_specs=[
            pl.BlockSpec((2, _BLK, 64), lambda i: (0, i, 0)),
            pl.BlockSpec((_BLK, 64), lambda i: (i, 0)),
            pl.BlockSpec((_BLK, 1), lambda i: (i, 0)),
            pl.BlockSpec((1, 64), lambda i: (0, 0)),
        ],
        out_specs=pl.BlockSpec((_BLK, 64), lambda i: (i, 0)),
        out_shape=jax.ShapeDtypeStruct((_N, 64), jnp.float32),
    )(S, P, dis, b)


def kernel(x, edge_index, W1, b1, W2, b2, W3, b3):
    ch_cs = (_E // _NS) // _K
    ch_es = (_E // (_NC * _NS)) // _K
    src_cs = edge_index[0].reshape(_NS, ch_cs, _K)
    dst_cs = edge_index[1].reshape(_NS, ch_cs, _K)
    src_es = edge_index[0].reshape(_NC * _NS, ch_es, _K)
    dst_es = edge_index[1].reshape(_NC * _NS, ch_es, _K)
    zeros64 = jnp.zeros((_ZROWS, 64), jnp.float32)
    zeros1d = jnp.zeros((_ZROWS,), jnp.float32)

    degS = _sc_deg(dst_es, zeros1d)
    dis, P1 = _tc_first(degS.T, x, W1)
    S1 = _sc_agg_cs(P1, src_cs, dst_cs, zeros64)
    P2 = _tc_mid(S1, P1, dis, b1.reshape(1, 128), W2, 128, split_out=True)
    S2 = _sc_agg_cs(P2, src_cs, dst_cs, zeros64)
    P3 = _tc_mid(S2, P2, dis, b2.reshape(1, 128), W3, 64, split_out=False)
    S3 = _sc_agg_64(P3, src_es, dst_es, zeros64)
    return _tc_final(S3, P3, dis, b3.reshape(1, 64))


# speedup vs baseline: 28.6981x; 1.0520x over previous
import functools

import jax
import jax.numpy as jnp
from jax import lax
from jax.experimental import pallas as pl
from jax.experimental.pallas import tpu as pltpu
from jax.experimental.pallas import tpu_sc as plsc

_N = 10000
_E = 320000
_K = 80
_NC = 2
_NS = 16
_NPAD = 10240
_ZROWS = _NPAD // _NS


def _make_sc_agg_colsplit(NBUF=5):
    mesh = plsc.VectorSubcoreMesh(core_axis_name="c", subcore_axis_name="s")
    pt = _E // _NS
    ch = pt // _K
    ng = ch // NBUF
    tail = ch - ng * NBUF

    @functools.partial(
        pl.kernel,
        out_type=jax.ShapeDtypeStruct((_NC, _NPAD, 64), jnp.float32),
        mesh=mesh,
        compiler_params=pltpu.CompilerParams(use_tc_tiling_on_sc=False),
        scratch_types=[
            pltpu.VMEM((ch, _K), jnp.int32),
            pltpu.VMEM((ch, _K), jnp.int32),
            pltpu.VMEM((NBUF, _K, 64), jnp.float32),
            pltpu.VMEM_SHARED((_NPAD, 64), jnp.float32),
            pltpu.SemaphoreType.DMA((NBUF,)),
            pltpu.SemaphoreType.DMA((NBUF,)),
        ],
    )
    def agg(p_hbm, src_hbm, dst_hbm, zeros_hbm, out_hbm, src_i, dst_i, rows, acc, gsem, ssem):
        c = lax.axis_index("c")
        s = lax.axis_index("s")
        pltpu.async_copy(zeros_hbm, acc.at[pl.ds(s * _ZROWS, _ZROWS)], gsem.at[0])
        pltpu.async_copy(src_hbm.at[s], src_i, gsem.at[1])
        pltpu.async_copy(dst_hbm.at[s], dst_i, gsem.at[2])
        pltpu.make_async_copy(zeros_hbm, acc.at[pl.ds(s * _ZROWS, _ZROWS)], gsem.at[0]).wait()
        pltpu.make_async_copy(src_hbm.at[s], src_i, gsem.at[1]).wait()
        pltpu.make_async_copy(dst_hbm.at[s], dst_i, gsem.at[2]).wait()
        plsc.subcore_barrier()
        pc = p_hbm.at[c]

        def gather(j, b):
            pltpu.async_copy(pc.at[src_i.at[j]], rows.at[b], gsem.at[b])

        def wait_gather(j, b):
            pltpu.make_async_copy(pc.at[src_i.at[j]], rows.at[b], gsem.at[b]).wait()

        def scatter(j, b):
            pltpu.async_copy(rows.at[b], acc.at[dst_i.at[j]], ssem.at[b], add=True)

        def wait_scatter(j, b):
            pltpu.make_async_copy(rows.at[b], acc.at[dst_i.at[j]], ssem.at[b]).wait()

        for b in range(NBUF):
            gather(b, b)

        def group(g, carry):
            for b in range(NBUF):
                j = g * NBUF + b
                wait_gather(j, b)
                scatter(j, b)
            for b in range(NBUF):
                j = g * NBUF + b
                wait_scatter(j, b)
                jn = (g + 1) * NBUF + b

                @pl.when(jn < ch)
                def _():
                    gather(jn, b)
            return carry

        lax.fori_loop(0, ng, group, 0)
        for b in range(tail):
            j = ng * NBUF + b
            wait_gather(j, b)
            scatter(j, b)
            wait_scatter(j, b)

        plsc.subcore_barrier()
        r0 = s * _ZROWS
        pltpu.sync_copy(acc.at[pl.ds(r0, _ZROWS)], out_hbm.at[c, pl.ds(r0, _ZROWS)])

    return agg


def _make_sc_agg_edgesplit64(NBUF=5):
    mesh = plsc.VectorSubcoreMesh(core_axis_name="c", subcore_axis_name="s")
    pt = _E // (_NC * _NS)
    ch = pt // _K
    ng = ch // NBUF
    tail = ch - ng * NBUF

    @functools.partial(
        pl.kernel,
        out_type=jax.ShapeDtypeStruct((_NC, _NPAD, 64), jnp.float32),
        mesh=mesh,
        compiler_params=pltpu.CompilerParams(use_tc_tiling_on_sc=False),
        scratch_types=[
            pltpu.VMEM((ch, _K), jnp.int32),
            pltpu.VMEM((ch, _K), jnp.int32),
            pltpu.VMEM((NBUF, _K, 64), jnp.float32),
            pltpu.VMEM_SHARED((_NPAD, 64), jnp.float32),
            pltpu.SemaphoreType.DMA((NBUF,)),
            pltpu.SemaphoreType.DMA((NBUF,)),
        ],
    )
    def agg(p_hbm, src_hbm, dst_hbm, zeros_hbm, out_hbm, src_i, dst_i, rows, acc, gsem, ssem):
        c = lax.axis_index("c")
        s = lax.axis_index("s")
        t = c * _NS + s
        pltpu.async_copy(zeros_hbm, acc.at[pl.ds(s * _ZROWS, _ZROWS)], gsem.at[0])
        pltpu.async_copy(src_hbm.at[t], src_i, gsem.at[1])
        pltpu.async_copy(dst_hbm.at[t], dst_i, gsem.at[2])
        pltpu.make_async_copy(zeros_hbm, acc.at[pl.ds(s * _ZROWS, _ZROWS)], gsem.at[0]).wait()
        pltpu.make_async_copy(src_hbm.at[t], src_i, gsem.at[1]).wait()
        pltpu.make_async_copy(dst_hbm.at[t], dst_i, gsem.at[2]).wait()
        plsc.subcore_barrier()

        def gather(j, b):
            pltpu.async_copy(p_hbm.at[src_i.at[j]], rows.at[b], gsem.at[b])

        def wait_gather(j, b):
            pltpu.make_async_copy(p_hbm.at[src_i.at[j]], rows.at[b], gsem.at[b]).wait()

        def scatter(j, b):
            pltpu.async_copy(rows.at[b], acc.at[dst_i.at[j]], ssem.at[b], add=True)

        def wait_scatter(j, b):
            pltpu.make_async_copy(rows.at[b], acc.at[dst_i.at[j]], ssem.at[b]).wait()

        for b in range(NBUF):
            gather(b, b)

        def group(g, carry):
            for b in range(NBUF):
                j = g * NBUF + b
                wait_gather(j, b)
                scatter(j, b)
            for b in range(NBUF):
                j = g * NBUF + b
                wait_scatter(j, b)
                jn = (g + 1) * NBUF + b

                @pl.when(jn < ch)
                def _():
                    gather(jn, b)
            return carry

        lax.fori_loop(0, ng, group, 0)
        for b in range(tail):
            j = ng * NBUF + b
            wait_gather(j, b)
            scatter(j, b)
            wait_scatter(j, b)

        plsc.subcore_barrier()
        r0 = s * _ZROWS
        pltpu.sync_copy(acc.at[pl.ds(r0, _ZROWS)], out_hbm.at[c, pl.ds(r0, _ZROWS)])

    return agg


def _make_sc_degree():
    mesh = plsc.VectorSubcoreMesh(core_axis_name="c", subcore_axis_name="s")
    pt = _E // (_NC * _NS)
    ch = pt // _K

    @functools.partial(
        pl.kernel,
        out_type=jax.ShapeDtypeStruct((_NC, _NPAD), jnp.float32),
        mesh=mesh,
        scratch_types=[
            pltpu.VMEM((ch, _K), jnp.int32),
            pltpu.VMEM((_K,), jnp.float32),
            pltpu.VMEM((_ZROWS,), jnp.float32),
            pltpu.VMEM_SHARED((_NPAD,), jnp.float32),
            pltpu.SemaphoreType.DMA((4,)),
        ],
    )
    def deg(dst_hbm, zeros_hbm, out_hbm, dst_i, ones_v, zv, acc, ssem):
        c = lax.axis_index("c")
        s = lax.axis_index("s")
        pltpu.sync_copy(zeros_hbm, zv)
        pltpu.sync_copy(zv, acc.at[pl.ds(s * _ZROWS, _ZROWS)])
        for i in range(_K // 16):
            ones_v[pl.ds(i * 16, 16)] = jnp.ones((16,), jnp.float32)
        t = c * _NS + s
        pltpu.sync_copy(dst_hbm.at[t], dst_i)
        plsc.subcore_barrier()

        nb = 4
        for j in range(ch):
            if j >= nb:
                pltpu.make_async_copy(ones_v, acc.at[dst_i.at[j - nb]],
                                      ssem.at[j % nb]).wait()
            pltpu.async_copy(ones_v, acc.at[dst_i.at[j]], ssem.at[j % nb], add=True)
        for j in range(ch - nb, ch):
            pltpu.make_async_copy(ones_v, acc.at[dst_i.at[j]],
                                  ssem.at[j % nb]).wait()
        plsc.subcore_barrier()
        r0 = s * _ZROWS
        pltpu.sync_copy(acc.at[pl.ds(r0, _ZROWS)], out_hbm.at[c, pl.ds(r0, _ZROWS)])

    return deg


_sc_agg_cs = _make_sc_agg_colsplit()
_sc_agg_64 = _make_sc_agg_edgesplit64()
_sc_deg = _make_sc_degree()

_BLK = 2000


def _tc_first(degT, x, W1):
    def body(deg_ref, x_ref, w_ref, dis_ref, p_ref):
        deg = deg_ref[:, 0:1] + deg_ref[:, 1:2] + 1.0
        d = lax.rsqrt(deg)
        dis_ref[...] = d
        h = jnp.dot(x_ref[...], w_ref[...], preferred_element_type=jnp.float32)
        p_ref[0] = d * h[:, :64]
        p_ref[1] = d * h[:, 64:]

    return pl.pallas_call(
        body,
        grid=(_N // _BLK,),
        in_specs=[
            pl.BlockSpec((_BLK, 2), lambda i: (i, 0)),
            pl.BlockSpec((_BLK, 128), lambda i: (i, 0)),
            pl.BlockSpec((128, 128), lambda i: (0, 0)),
        ],
        out_specs=[
            pl.BlockSpec((_BLK, 1), lambda i: (i, 0)),
            pl.BlockSpec((2, _BLK, 64), lambda i: (0, i, 0)),
        ],
        out_shape=[
            jax.ShapeDtypeStruct((_N, 1), jnp.float32),
            jax.ShapeDtypeStruct((2, _N, 64), jnp.float32),
        ],
    )(degT, x, W1)


def _tc_mid(S, P, dis, b, W, Dn, split_out):
    def body(s_ref, p_ref, dis_ref, b_ref, w_ref, out_ref):
        d = dis_ref[...]
        h0 = jnp.maximum(d * (s_ref[0] + p_ref[0]) + b_ref[:, :64], 0.0)
        h1 = jnp.maximum(d * (s_ref[1] + p_ref[1]) + b_ref[:, 64:], 0.0)
        hw = (jnp.dot(h0, w_ref[:64], preferred_element_type=jnp.float32)
              + jnp.dot(h1, w_ref[64:], preferred_element_type=jnp.float32))
        if split_out:
            out_ref[0] = d * hw[:, : Dn // 2]
            out_ref[1] = d * hw[:, Dn // 2:]
        else:
            out_ref[...] = d * hw

    if split_out:
        out_spec = pl.BlockSpec((2, _BLK, Dn // 2), lambda i: (0, i, 0))
        out_shape = jax.ShapeDtypeStruct((2, _N, Dn // 2), jnp.float32)
    else:
        out_spec = pl.BlockSpec((_BLK, Dn), lambda i: (i, 0))
        out_shape = jax.ShapeDtypeStruct((_N, Dn), jnp.float32)

    return pl.pallas_call(
        body,
        grid=(_N // _BLK,),
        in_specs=[
            pl.BlockSpec((2, _BLK, 64), lambda i: (0, i, 0)),
            pl.BlockSpec((2, _BLK, 64), lambda i: (0, i, 0)),
            pl.BlockSpec((_BLK, 1), lambda i: (i, 0)),
            pl.BlockSpec((1, 128), lambda i: (0, 0)),
            pl.BlockSpec((128, Dn), lambda i: (0, 0)),
        ],
        out_specs=out_spec,
        out_shape=out_shape,
    )(S, P, dis, b, W)


def _tc_final(S, P, dis, b):
    def body(s_ref, p_ref, dis_ref, b_ref, out_ref):
        d = dis_ref[...]
        h = d * (s_ref[0] + s_ref[1] + p_ref[...]) + b_ref[...]
        m = jnp.max(h, axis=1, keepdims=True)
        sh = h - m
        out_ref[...] = sh - jnp.log(jnp.sum(jnp.exp(sh), axis=1, keepdims=True))

    return pl.pallas_call(
        body,
        grid=(_N // _BLK,),
        in_specs=[
            pl.BlockSpec((2, _BLK, 64), lambda i: (0, i, 0)),
            pl.BlockSpec((_BLK, 64), lambda i: (i, 0)),
            pl.BlockSpec((_BLK, 1), lambda i: (i, 0)),
            pl.BlockSpec((1, 64), lambda i: (0, 0)),
        ],
        out_specs=pl.BlockSpec((_BLK, 64), lambda i: (i, 0)),
        out_shape=jax.ShapeDtypeStruct((_N, 64), jnp.float32),
    )(S, P, dis, b)


def kernel(x, edge_index, W1, b1, W2, b2, W3, b3):
    ch_cs = (_E // _NS) // _K
    ch_es = (_E // (_NC * _NS)) // _K
    src_cs = edge_index[0].reshape(_NS, ch_cs, _K)
    dst_cs = edge_index[1].reshape(_NS, ch_cs, _K)
    src_es = edge_index[0].reshape(_NC * _NS, ch_es, _K)
    dst_es = edge_index[1].reshape(_NC * _NS, ch_es, _K)
    zeros64 = jnp.zeros((_ZROWS, 64), jnp.float32)
    zeros1d = jnp.zeros((_ZROWS,), jnp.float32)

    degS = _sc_deg(dst_es, zeros1d)
    dis, P1 = _tc_first(degS.T, x, W1)
    S1 = _sc_agg_cs(P1, src_cs, dst_cs, zeros64)
    P2 = _tc_mid(S1, P1, dis, b1.reshape(1, 128), W2, 128, split_out=True)
    S2 = _sc_agg_cs(P2, src_cs, dst_cs, zeros64)
    P3 = _tc_mid(S2, P2, dis, b2.reshape(1, 128), W3, 64, split_out=False)
    S3 = _sc_agg_64(P3, src_es, dst_es, zeros64)
    return _tc_final(S3, P3, dis, b3.reshape(1, 64))
